# baseline (speedup 1.0000x reference)
import functools

import jax
import jax.numpy as jnp
from jax import lax
from jax.experimental import pallas as pl
from jax.experimental.pallas import tpu as pltpu
from jax.experimental.pallas import tpu_sc as plsc

B, S, D = 4096, 200, 128
NC, NS, L = 2, 16, 16
NW = NC * NS
BPW = B // NW
HALF = 104
DW = D // 2
NCH = DW // L

_mesh = plsc.VectorSubcoreMesh(core_axis_name="c", subcore_axis_name="s")


def _fire(table_hbm, tok_v, rbuf, sem, i):
    pltpu.async_copy(table_hbm.at[tok_v.at[pl.ds(i * S, HALF)]],
                     rbuf.at[pl.ds(0, HALF)], sem)
    pltpu.async_copy(table_hbm.at[tok_v.at[pl.ds(i * S + HALF, S - HALF)]],
                     rbuf.at[pl.ds(HALF, S - HALF)], sem)


def _wait(table_hbm, tok_v, rbuf, sem, i):
    pltpu.make_async_copy(table_hbm.at[tok_v.at[pl.ds(i * S, HALF)]],
                          rbuf.at[pl.ds(0, HALF)], sem).wait()
    pltpu.make_async_copy(table_hbm.at[tok_v.at[pl.ds(i * S + HALF, S - HALF)]],
                          rbuf.at[pl.ds(HALF, S - HALF)], sem).wait()


def _accumulate(rbuf, out_v, i):
    def acc_body(r, accs):
        accs = list(accs)
        for c in range(NCH):
            w = rbuf[r, pl.ds(c * L, L)]
            lo = lax.bitcast_convert_type(
                lax.shift_left(w, 16), jnp.float32)
            hi = lax.bitcast_convert_type(
                lax.bitwise_and(w, jnp.int32(-65536)), jnp.float32)
            accs[c] = accs[c] + lo
            accs[NCH + c] = accs[NCH + c] + hi
        return tuple(accs)

    accs = lax.fori_loop(
        0, S, acc_body,
        tuple(jnp.zeros((L,), jnp.float32) for _ in range(2 * NCH)))

    for c in range(NCH):
        out_v[i, pl.ds(c * L, L)] = accs[c] * (1.0 / S)
        out_v[i, pl.ds(D // 2 + c * L, L)] = accs[NCH + c] * (1.0 / S)


@functools.partial(
    pl.kernel,
    mesh=_mesh,
    out_type=jax.ShapeDtypeStruct((B, D), jnp.float32),
    compiler_params=pltpu.CompilerParams(use_tc_tiling_on_sc=False),
    scratch_types=[
        pltpu.VMEM((BPW * S,), jnp.int32),
        pltpu.VMEM((S, DW), jnp.int32),
        pltpu.VMEM((S, DW), jnp.int32),
        pltpu.VMEM((BPW, D), jnp.float32),
        pltpu.SemaphoreType.DMA,
        pltpu.SemaphoreType.DMA,
    ],
)
def _embed_mean(tokens_hbm, table_hbm, out_hbm,
                tok_v, buf0, buf1, out_v, sem0, sem1):
    wid = lax.axis_index("s") * NC + lax.axis_index("c")
    base = wid * BPW
    pltpu.sync_copy(tokens_hbm.at[pl.ds(base * S, BPW * S)], tok_v)

    _fire(table_hbm, tok_v, buf0, sem0, 0)
    _fire(table_hbm, tok_v, buf1, sem1, 1)

    def pair_body(j, carry):
        i2 = j * 2
        _wait(table_hbm, tok_v, buf0, sem0, i2)
        _accumulate(buf0, out_v, i2)

        @pl.when(i2 + 2 < BPW)
        def _():
            _fire(table_hbm, tok_v, buf0, sem0, i2 + 2)

        _wait(table_hbm, tok_v, buf1, sem1, i2 + 1)
        _accumulate(buf1, out_v, i2 + 1)

        @pl.when(i2 + 3 < BPW)
        def _():
            _fire(table_hbm, tok_v, buf1, sem1, i2 + 3)

        return carry

    lax.fori_loop(0, BPW // 2, pair_body, 0)
    pltpu.sync_copy(out_v, out_hbm.at[pl.ds(base, BPW)])


def kernel(tokens, table):
    tok_flat = tokens.reshape(-1).astype(jnp.int32)
    tb = table.astype(jnp.bfloat16)
    tw = lax.bitcast_convert_type(
        jnp.stack([tb[:, : D // 2], tb[:, D // 2:]], axis=2), jnp.int32)
    return _embed_mean(tok_flat, tw)


# speedup vs baseline: 1.0383x; 1.0383x over previous
import functools

import jax
import jax.numpy as jnp
from jax import lax
from jax.experimental import pallas as pl
from jax.experimental.pallas import tpu as pltpu
from jax.experimental.pallas import tpu_sc as plsc

B, S, D = 4096, 200, 128
NC, NS, L = 2, 16, 16
NW = NC * NS
BPW = B // NW
HALF = 104
DW = D // 2
NCH = DW // L

_mesh = plsc.VectorSubcoreMesh(core_axis_name="c", subcore_axis_name="s")


def _fire(table_hbm, tok_v, rbuf, sem, i):
    pltpu.async_copy(table_hbm.at[tok_v.at[pl.ds(i * S, HALF)]],
                     rbuf.at[pl.ds(0, HALF)], sem)
    pltpu.async_copy(table_hbm.at[tok_v.at[pl.ds(i * S + HALF, S - HALF)]],
                     rbuf.at[pl.ds(HALF, S - HALF)], sem)


def _wait(table_hbm, tok_v, rbuf, sem, i):
    pltpu.make_async_copy(table_hbm.at[tok_v.at[pl.ds(i * S, HALF)]],
                          rbuf.at[pl.ds(0, HALF)], sem).wait()
    pltpu.make_async_copy(table_hbm.at[tok_v.at[pl.ds(i * S + HALF, S - HALF)]],
                          rbuf.at[pl.ds(HALF, S - HALF)], sem).wait()


def _accumulate(rbuf, out_v, i):
    def acc_body(q, accs):
        accs = list(accs)
        r = q * 4
        for u in range(4):
            for c in range(NCH):
                w = rbuf[r + u, pl.ds(c * L, L)]
                lo = lax.bitcast_convert_type(
                    lax.shift_left(w, 16), jnp.float32)
                hi = lax.bitcast_convert_type(w, jnp.float32)
                accs[c] = accs[c] + lo
                accs[NCH + c] = accs[NCH + c] + hi
        return tuple(accs)

    accs = lax.fori_loop(
        0, S // 4, acc_body,
        tuple(jnp.zeros((L,), jnp.float32) for _ in range(2 * NCH)))

    for c in range(NCH):
        out_v[i, pl.ds(c * L, L)] = accs[c] * (1.0 / S)
        out_v[i, pl.ds(D // 2 + c * L, L)] = accs[NCH + c] * (1.0 / S)


@functools.partial(
    pl.kernel,
    mesh=_mesh,
    out_type=jax.ShapeDtypeStruct((B, D), jnp.float32),
    compiler_params=pltpu.CompilerParams(use_tc_tiling_on_sc=False),
    scratch_types=[
        pltpu.VMEM((BPW * S,), jnp.int32),
        pltpu.VMEM((S, DW), jnp.int32),
        pltpu.VMEM((S, DW), jnp.int32),
        pltpu.VMEM((BPW, D), jnp.float32),
        pltpu.SemaphoreType.DMA,
        pltpu.SemaphoreType.DMA,
    ],
)
def _embed_mean(tokens_hbm, table_hbm, out_hbm,
                tok_v, buf0, buf1, out_v, sem0, sem1):
    wid = lax.axis_index("s") * NC + lax.axis_index("c")
    base = wid * BPW
    pltpu.sync_copy(tokens_hbm.at[pl.ds(base * S, BPW * S)], tok_v)

    _fire(table_hbm, tok_v, buf0, sem0, 0)
    _fire(table_hbm, tok_v, buf1, sem1, 1)

    def pair_body(j, carry):
        i2 = j * 2
        _wait(table_hbm, tok_v, buf0, sem0, i2)
        _accumulate(buf0, out_v, i2)

        @pl.when(i2 + 2 < BPW)
        def _():
            _fire(table_hbm, tok_v, buf0, sem0, i2 + 2)

        _wait(table_hbm, tok_v, buf1, sem1, i2 + 1)
        _accumulate(buf1, out_v, i2 + 1)

        @pl.when(i2 + 3 < BPW)
        def _():
            _fire(table_hbm, tok_v, buf1, sem1, i2 + 3)

        return carry

    lax.fori_loop(0, BPW // 2, pair_body, 0)
    pltpu.sync_copy(out_v, out_hbm.at[pl.ds(base, BPW)])


def kernel(tokens, table):
    tok_flat = tokens.reshape(-1).astype(jnp.int32)
    tb = table.astype(jnp.bfloat16)
    tw = lax.bitcast_convert_type(
        jnp.stack([tb[:, : D // 2], tb[:, D // 2:]], axis=2), jnp.int32)
    return _embed_mean(tok_flat, tw)


# speedup vs baseline: 1.2170x; 1.1721x over previous
import functools

import jax
import jax.numpy as jnp
from jax import lax
from jax.experimental import pallas as pl
from jax.experimental.pallas import tpu as pltpu
from jax.experimental.pallas import tpu_sc as plsc

B, S, D = 4096, 200, 128
NC, NS, L = 2, 16, 16
NW = NC * NS
BPW = B // NW
HALF = 104
DW = D // 2
NCH = DW // L

_mesh = plsc.VectorSubcoreMesh(core_axis_name="c", subcore_axis_name="s")


def _fire(table_hbm, tok_v, rbuf, sem, i):
    pltpu.async_copy(table_hbm.at[tok_v.at[pl.ds(i * S, HALF)]],
                     rbuf.at[pl.ds(0, HALF)], sem)
    pltpu.async_copy(table_hbm.at[tok_v.at[pl.ds(i * S + HALF, S - HALF)]],
                     rbuf.at[pl.ds(HALF, S - HALF)], sem)


def _wait(table_hbm, tok_v, rbuf, sem, i):
    pltpu.make_async_copy(table_hbm.at[tok_v.at[pl.ds(i * S, HALF)]],
                          rbuf.at[pl.ds(0, HALF)], sem).wait()
    pltpu.make_async_copy(table_hbm.at[tok_v.at[pl.ds(i * S + HALF, S - HALF)]],
                          rbuf.at[pl.ds(HALF, S - HALF)], sem).wait()


def _accumulate(rbuf, out_v, i):
    def acc_body(q, accs):
        accs = list(accs)
        r = q * 4
        for u in range(4):
            for c in range(NCH):
                w = rbuf[r + u, pl.ds(c * L, L)]
                lo = lax.bitcast_convert_type(
                    lax.shift_left(w, 16), jnp.float32)
                hi = lax.bitcast_convert_type(w, jnp.float32)
                accs[c] = accs[c] + lo
                accs[NCH + c] = accs[NCH + c] + hi
        return tuple(accs)

    accs = lax.fori_loop(
        0, S // 4, acc_body,
        tuple(jnp.zeros((L,), jnp.float32) for _ in range(2 * NCH)))

    for c in range(NCH):
        out_v[i, pl.ds(c * L, L)] = accs[c] * (1.0 / S)
        out_v[i, pl.ds(D // 2 + c * L, L)] = accs[NCH + c] * (1.0 / S)


@functools.partial(
    pl.kernel,
    mesh=_mesh,
    out_type=jax.ShapeDtypeStruct((B, D), jnp.float32),
    compiler_params=pltpu.CompilerParams(use_tc_tiling_on_sc=False),
    scratch_types=[
        pltpu.VMEM((BPW * S,), jnp.int32),
        pltpu.VMEM((S, DW), jnp.int32),
        pltpu.VMEM((S, DW), jnp.int32),
        pltpu.VMEM((BPW, D), jnp.float32),
        pltpu.SemaphoreType.DMA,
        pltpu.SemaphoreType.DMA,
    ],
)
def _embed_mean(tokens_hbm, table_hbm, out_hbm,
                tok_v, buf0, buf1, out_v, sem0, sem1):
    wid = lax.axis_index("s") * NC + lax.axis_index("c")
    base = wid * BPW
    pltpu.sync_copy(tokens_hbm.at[pl.ds(base * S, BPW * S)], tok_v)

    _fire(table_hbm, tok_v, buf0, sem0, 0)
    _fire(table_hbm, tok_v, buf1, sem1, 1)

    def pair_body(j, carry):
        i2 = j * 2
        _wait(table_hbm, tok_v, buf0, sem0, i2)
        _accumulate(buf0, out_v, i2)

        @pl.when(i2 + 2 < BPW)
        def _():
            _fire(table_hbm, tok_v, buf0, sem0, i2 + 2)

        _wait(table_hbm, tok_v, buf1, sem1, i2 + 1)
        _accumulate(buf1, out_v, i2 + 1)

        @pl.when(i2 + 3 < BPW)
        def _():
            _fire(table_hbm, tok_v, buf1, sem1, i2 + 3)

        return carry

    lax.fori_loop(0, BPW // 2, pair_body, 0)
    pltpu.sync_copy(out_v, out_hbm.at[pl.ds(base, BPW)])


def kernel(tokens, table):
    tok_flat = tokens.reshape(-1).astype(jnp.int32)
    t32 = lax.bitcast_convert_type(table, jnp.int32)
    lo = lax.shift_right_logical(
        lax.shift_right_logical(t32[:, : D // 2], 15) + 1, 1)
    hi = lax.bitwise_and(t32[:, D // 2:] + 0x8000, jnp.int32(-65536))
    tw = lax.bitwise_or(hi, lo)
    return _embed_mean(tok_flat, tw)


# speedup vs baseline: 1.9370x; 1.5916x over previous
import functools

import jax
import jax.numpy as jnp
from jax import lax
from jax.experimental import pallas as pl
from jax.experimental.pallas import tpu as pltpu
from jax.experimental.pallas import tpu_sc as plsc

B, S, D = 4096, 200, 128
V = 100000
NC, NS, L = 2, 16, 16
NW = NC * NS
BPW = B // NW
HALF = 104
DW = D // 2
NCH = DW // L
VPW = V // NW
PCHUNK = 125

_mesh = plsc.VectorSubcoreMesh(core_axis_name="c", subcore_axis_name="s")
_params = pltpu.CompilerParams(use_tc_tiling_on_sc=False)


@functools.partial(
    pl.kernel,
    mesh=_mesh,
    out_type=jax.ShapeDtypeStruct((V, DW), jnp.int32),
    compiler_params=_params,
    scratch_types=[
        pltpu.VMEM((PCHUNK, D), jnp.float32),
        pltpu.VMEM((PCHUNK, D), jnp.float32),
        pltpu.VMEM((PCHUNK, DW), jnp.int32),
        pltpu.VMEM((PCHUNK, DW), jnp.int32),
        pltpu.SemaphoreType.DMA,
        pltpu.SemaphoreType.DMA,
    ],
)
def _pack_table(table_hbm, out_hbm, in0, in1, pk0, pk1, sem0, sem1):
    wid = lax.axis_index("s") * NC + lax.axis_index("c")
    base = wid * VPW
    nstep = VPW // PCHUNK

    def fetch(step, buf, sem):
        pltpu.async_copy(
            table_hbm.at[pl.ds(base + step * PCHUNK, PCHUNK)], buf, sem)

    def fwait(buf, sem):
        pltpu.make_async_copy(table_hbm.at[pl.ds(0, PCHUNK)], buf, sem).wait()

    def pack_chunk(step, buf, pk, sem):
        def row_body(r, carry):
            for c in range(NCH):
                flo = buf[r, pl.ds(c * L, L)]
                fhi = buf[r, pl.ds(DW + c * L, L)]
                ilo = lax.bitcast_convert_type(flo, jnp.int32)
                ihi = lax.bitcast_convert_type(fhi, jnp.int32)
                lo = lax.shift_right_logical(
                    lax.shift_right_logical(ilo, 15) + 1, 1)
                hi = lax.bitwise_and(ihi + 0x8000, jnp.int32(-65536))
                pk[r, pl.ds(c * L, L)] = lax.bitwise_or(hi, lo)
            return carry

        lax.fori_loop(0, PCHUNK, row_body, 0)
        pltpu.async_copy(
            pk, out_hbm.at[pl.ds(base + step * PCHUNK, PCHUNK)], sem)

    fetch(0, in0, sem0)
    fetch(1, in1, sem1)

    def pair_body(j, carry):
        s2 = j * 2
        fwait(in0, sem0)
        pack_chunk(s2, in0, pk0, sem0)

        @pl.when(s2 + 2 < nstep)
        def _():
            fetch(s2 + 2, in0, sem0)

        fwait(in1, sem1)
        pack_chunk(s2 + 1, in1, pk1, sem1)

        @pl.when(s2 + 3 < nstep)
        def _():
            fetch(s2 + 3, in1, sem1)

        return carry

    lax.fori_loop(0, nstep // 2, pair_body, 0)
    pltpu.make_async_copy(pk0, out_hbm.at[pl.ds(0, PCHUNK)], sem0).wait()
    pltpu.make_async_copy(pk1, out_hbm.at[pl.ds(0, PCHUNK)], sem1).wait()


def _fire(table_hbm, tok_v, rbuf, sem, i):
    pltpu.async_copy(table_hbm.at[tok_v.at[pl.ds(i * S, HALF)]],
                     rbuf.at[pl.ds(0, HALF)], sem)
    pltpu.async_copy(table_hbm.at[tok_v.at[pl.ds(i * S + HALF, S - HALF)]],
                     rbuf.at[pl.ds(HALF, S - HALF)], sem)


def _wait(table_hbm, tok_v, rbuf, sem, i):
    pltpu.make_async_copy(table_hbm.at[tok_v.at[pl.ds(i * S, HALF)]],
                          rbuf.at[pl.ds(0, HALF)], sem).wait()
    pltpu.make_async_copy(table_hbm.at[tok_v.at[pl.ds(i * S + HALF, S - HALF)]],
                          rbuf.at[pl.ds(HALF, S - HALF)], sem).wait()


def _accumulate(rbuf, out_v, i):
    def acc_body(q, accs):
        accs = list(accs)
        r = q * 4
        for u in range(4):
            for c in range(NCH):
                w = rbuf[r + u, pl.ds(c * L, L)]
                lo = lax.bitcast_convert_type(
                    lax.shift_left(w, 16), jnp.float32)
                hi = lax.bitcast_convert_type(w, jnp.float32)
                accs[c] = accs[c] + lo
                accs[NCH + c] = accs[NCH + c] + hi
        return tuple(accs)

    accs = lax.fori_loop(
        0, S // 4, acc_body,
        tuple(jnp.zeros((L,), jnp.float32) for _ in range(2 * NCH)))

    for c in range(NCH):
        out_v[i, pl.ds(c * L, L)] = accs[c] * (1.0 / S)
        out_v[i, pl.ds(D // 2 + c * L, L)] = accs[NCH + c] * (1.0 / S)


@functools.partial(
    pl.kernel,
    mesh=_mesh,
    out_type=jax.ShapeDtypeStruct((B, D), jnp.float32),
    compiler_params=_params,
    scratch_types=[
        pltpu.VMEM((BPW * S,), jnp.int32),
        pltpu.VMEM((S, DW), jnp.int32),
        pltpu.VMEM((S, DW), jnp.int32),
        pltpu.VMEM((BPW, D), jnp.float32),
        pltpu.SemaphoreType.DMA,
        pltpu.SemaphoreType.DMA,
    ],
)
def _embed_mean(tokens_hbm, table_hbm, out_hbm,
                tok_v, buf0, buf1, out_v, sem0, sem1):
    wid = lax.axis_index("s") * NC + lax.axis_index("c")
    base = wid * BPW
    pltpu.sync_copy(tokens_hbm.at[pl.ds(base * S, BPW * S)], tok_v)

    _fire(table_hbm, tok_v, buf0, sem0, 0)
    _fire(table_hbm, tok_v, buf1, sem1, 1)

    def pair_body(j, carry):
        i2 = j * 2
        _wait(table_hbm, tok_v, buf0, sem0, i2)
        _accumulate(buf0, out_v, i2)

        @pl.when(i2 + 2 < BPW)
        def _():
            _fire(table_hbm, tok_v, buf0, sem0, i2 + 2)

        _wait(table_hbm, tok_v, buf1, sem1, i2 + 1)
        _accumulate(buf1, out_v, i2 + 1)

        @pl.when(i2 + 3 < BPW)
        def _():
            _fire(table_hbm, tok_v, buf1, sem1, i2 + 3)

        return carry

    lax.fori_loop(0, BPW // 2, pair_body, 0)
    pltpu.sync_copy(out_v, out_hbm.at[pl.ds(base, BPW)])


def kernel(tokens, table):
    tok_flat = tokens.reshape(-1).astype(jnp.int32)
    tw = _pack_table(table)
    return _embed_mean(tok_flat, tw)
